# baseline (speedup 1.0000x reference)
import jax
import jax.numpy as jnp
from jax.experimental import pallas as pl
from jax.experimental.pallas import tpu as pltpu

_L = 4
_B = 1024
_D = 1024
_EPS = 1e-6
_E0 = 1.0
_TILE = 128
_T = _D // _TILE
_FC = 256


def _body(x0, k1, k1n, k2, k3, k3n, k4, ta0, ti0, cin0, masks,
          k5, k5n, k6, kdi, kdt, out_ref,
          x_bf, ai, cc, v_ref):
    l = pl.program_id(0)
    t = pl.program_id(1)

    @pl.when(jnp.logical_and(l == 0, t == 0))
    def _init():
        out_ref[:] = jnp.ones_like(out_ref)
        x_bf[:] = x0[:].astype(jnp.bfloat16)

    @pl.when(t < _T)
    def _tile():
        m = masks[0]
        kact = jnp.where(
            m > 0,
            ta0[0] * k1[0] * pl.reciprocal(k1n[0] + k2[0] + _EPS,
                                           approx=True, full_range=False),
            0.0)
        kinh = jnp.where(
            m < 0,
            ti0[0] * k3[0] * pl.reciprocal(k3n[0] + k4[0] + _EPS,
                                           approx=True, full_range=False),
            0.0)
        cc[0:_TILE, :] = (k2[0] * kact).astype(jnp.bfloat16)
        cc[_TILE:2 * _TILE, :] = (cin0[0] * k4[0] * kinh).astype(
            jnp.bfloat16)
        colsum = jnp.sum(kact + kinh, axis=0, keepdims=True)

        @pl.when(t == 0)
        def _():
            v_ref[:] = colsum

        @pl.when(t > 0)
        def _():
            v_ref[:] = v_ref[:] + colsum

        dn = (((1,), (1,)), ((), ()))
        ab = jax.lax.dot_general(x_bf[:], cc[:], dn,
                                 preferred_element_type=jnp.float32)
        ai[:, pl.ds(2 * _TILE * t, 2 * _TILE)] = ab.astype(jnp.bfloat16)

    @pl.when(t == _T)
    def _finalize():
        acc = jnp.zeros((_B, 1), jnp.float32)
        for c in range(_D // _FC):
            sl = slice(c * _FC, (c + 1) * _FC)
            acc += jnp.sum(x_bf[:, sl].astype(jnp.float32) * v_ref[:, sl],
                           axis=1, keepdims=True)
        cp = out_ref[:] + acc
        cp2 = jnp.zeros((_B, 1), jnp.float32)
        for c in range(_D // _FC):
            sl = slice(c * _FC, (c + 1) * _FC)
            activ = ai[:, 2 * _FC * c:2 * _FC * c + _FC].astype(
                jnp.float32)
            inhib = ai[:, 2 * _FC * c + _FC:2 * _FC * (c + 1)].astype(
                jnp.float32)
            kdicp = kdi[0, :, sl] * cp
            num = _E0 * activ * kdicp
            den = kdt[0, :, sl] * kdicp * cp + _E0 * inhib + _EPS * kdicp
            x_eq = num * pl.reciprocal(den, approx=True, full_range=False)
            w5 = k5[0, :, sl] * pl.reciprocal(
                k5n[0, :, sl] + k6[0, :, sl] + _EPS,
                approx=True, full_range=False)
            cp2 += jnp.sum(x_eq * w5, axis=1, keepdims=True)
            x_bf[:, sl] = x_eq.astype(jnp.bfloat16)
        out_ref[:] = cp + cp2


def kernel(X0, k1, k1n, k2, k3, k3n, k4, TA0, TI0, Cinhib0, masks,
           k5, k5n, k6, kdI, kdT):
    big = pl.BlockSpec((1, _TILE, _D),
                       lambda l, t: (l, jnp.minimum(t, _T - 1), 0))
    vec = pl.BlockSpec((1, 1, _D), lambda l, t: (l, 0, 0))
    k5, k5n, k6, kdI, kdT = (a.reshape(_L, 1, _D)
                             for a in (k5, k5n, k6, kdI, kdT))
    cp = pl.pallas_call(
        _body,
        grid=(_L, _T + 1),
        in_specs=[pl.BlockSpec((_B, _D), lambda l, t: (0, 0))]
        + [big] * 10 + [vec] * 5,
        out_specs=pl.BlockSpec((_B, 1), lambda l, t: (0, 0)),
        out_shape=jax.ShapeDtypeStruct((_B, 1), jnp.float32),
        scratch_shapes=[
            pltpu.VMEM((_B, _D), jnp.bfloat16),
            pltpu.VMEM((_B, 2 * _D), jnp.bfloat16),
            pltpu.VMEM((2 * _TILE, _D), jnp.bfloat16),
            pltpu.VMEM((1, _D), jnp.float32),
        ],
        compiler_params=pltpu.CompilerParams(
            vmem_limit_bytes=100 * 1024 * 1024),
    )(X0, k1, k1n, k2, k3, k3n, k4, TA0, TI0, Cinhib0, masks,
      k5, k5n, k6, kdI, kdT)
    return cp.reshape(_B)


# speedup vs baseline: 1.1079x; 1.1079x over previous
import jax
import jax.numpy as jnp
from jax.experimental import pallas as pl
from jax.experimental.pallas import tpu as pltpu

_L = 4
_B = 1024
_D = 1024
_EPS = 1e-6
_E0 = 1.0
_TILE = 256
_T = _D // _TILE
_FC = 256


def _body(x0, k1, k1n, k2, k3, k3n, k4, ta0, ti0, cin0, masks,
          k5, k5n, k6, kdi, kdt, out_ref,
          x_bf, ai, cc, v_ref):
    l = pl.program_id(0)
    t = pl.program_id(1)

    @pl.when(jnp.logical_and(l == 0, t == 0))
    def _init():
        out_ref[:] = jnp.ones_like(out_ref)
        x_bf[:] = x0[:].astype(jnp.bfloat16)

    @pl.when(t < _T)
    def _tile():
        m = masks[0]
        kact = jnp.where(
            m > 0,
            ta0[0] * k1[0] * pl.reciprocal(k1n[0] + k2[0] + _EPS,
                                           approx=True, full_range=False),
            0.0)
        kinh = jnp.where(
            m < 0,
            ti0[0] * k3[0] * pl.reciprocal(k3n[0] + k4[0] + _EPS,
                                           approx=True, full_range=False),
            0.0)
        cc[0:_TILE, :] = (k2[0] * kact).astype(jnp.bfloat16)
        cc[_TILE:2 * _TILE, :] = (cin0[0] * k4[0] * kinh).astype(
            jnp.bfloat16)
        colsum = jnp.sum(kact + kinh, axis=0, keepdims=True)

        @pl.when(t == 0)
        def _():
            v_ref[:] = colsum

        @pl.when(t > 0)
        def _():
            v_ref[:] = v_ref[:] + colsum

        dn = (((1,), (1,)), ((), ()))
        ab = jax.lax.dot_general(x_bf[:], cc[:], dn,
                                 preferred_element_type=jnp.float32)
        ai[:, pl.ds(2 * _TILE * t, 2 * _TILE)] = ab.astype(jnp.bfloat16)

    @pl.when(t == _T)
    def _finalize():
        acc = jnp.zeros((_B, 1), jnp.float32)
        for c in range(_D // _FC):
            sl = slice(c * _FC, (c + 1) * _FC)
            acc += jnp.sum(x_bf[:, sl].astype(jnp.float32) * v_ref[:, sl],
                           axis=1, keepdims=True)
        cp = out_ref[:] + acc
        cp2 = jnp.zeros((_B, 1), jnp.float32)
        for c in range(_D // _FC):
            sl = slice(c * _FC, (c + 1) * _FC)
            activ = ai[:, 2 * _FC * c:2 * _FC * c + _FC].astype(
                jnp.float32)
            inhib = ai[:, 2 * _FC * c + _FC:2 * _FC * (c + 1)].astype(
                jnp.float32)
            kdicp = kdi[0, :, sl] * cp
            num = _E0 * activ * kdicp
            den = kdt[0, :, sl] * kdicp * cp + _E0 * inhib + _EPS * kdicp
            x_eq = num * pl.reciprocal(den, approx=True, full_range=False)
            w5 = k5[0, :, sl] * pl.reciprocal(
                k5n[0, :, sl] + k6[0, :, sl] + _EPS,
                approx=True, full_range=False)
            cp2 += jnp.sum(x_eq * w5, axis=1, keepdims=True)
            x_bf[:, sl] = x_eq.astype(jnp.bfloat16)
        out_ref[:] = cp + cp2


def kernel(X0, k1, k1n, k2, k3, k3n, k4, TA0, TI0, Cinhib0, masks,
           k5, k5n, k6, kdI, kdT):
    big = pl.BlockSpec((1, _TILE, _D),
                       lambda l, t: (l, jnp.minimum(t, _T - 1), 0))
    vec = pl.BlockSpec((1, 1, _D), lambda l, t: (l, 0, 0))
    k5, k5n, k6, kdI, kdT = (a.reshape(_L, 1, _D)
                             for a in (k5, k5n, k6, kdI, kdT))
    cp = pl.pallas_call(
        _body,
        grid=(_L, _T + 1),
        in_specs=[pl.BlockSpec((_B, _D), lambda l, t: (0, 0))]
        + [big] * 10 + [vec] * 5,
        out_specs=pl.BlockSpec((_B, 1), lambda l, t: (0, 0)),
        out_shape=jax.ShapeDtypeStruct((_B, 1), jnp.float32),
        scratch_shapes=[
            pltpu.VMEM((_B, _D), jnp.bfloat16),
            pltpu.VMEM((_B, 2 * _D), jnp.bfloat16),
            pltpu.VMEM((2 * _TILE, _D), jnp.bfloat16),
            pltpu.VMEM((1, _D), jnp.float32),
        ],
        compiler_params=pltpu.CompilerParams(
            vmem_limit_bytes=100 * 1024 * 1024),
    )(X0, k1, k1n, k2, k3, k3n, k4, TA0, TI0, Cinhib0, masks,
      k5, k5n, k6, kdI, kdT)
    return cp.reshape(_B)


# speedup vs baseline: 1.1109x; 1.0026x over previous
import jax
import jax.numpy as jnp
from jax.experimental import pallas as pl
from jax.experimental.pallas import tpu as pltpu

_L = 4
_B = 1024
_D = 1024
_EPS = 1e-6
_E0 = 1.0
_TILE = 256
_T = _D // _TILE
_FC = 256


def _body(x0, k1, k1n, k2, k3, k3n, k4, ta0, ti0, cin0, masks,
          k5, k5n, k6, kdi, kdt, out_ref,
          x_bf, ai, cc, v_ref):
    l = pl.program_id(0)
    t = pl.program_id(1)

    @pl.when(jnp.logical_and(l == 0, t == 0))
    def _init():
        out_ref[:] = jnp.ones_like(out_ref)
        x_bf[:] = x0[:].astype(jnp.bfloat16)

    @pl.when(t < _T)
    def _tile():
        m = masks[0]
        kact = jnp.where(
            m > 0,
            ta0[0] * k1[0] * pl.reciprocal(k1n[0] + k2[0] + _EPS,
                                           approx=True, full_range=False),
            0.0)
        kinh = jnp.where(
            m < 0,
            ti0[0] * k3[0] * pl.reciprocal(k3n[0] + k4[0] + _EPS,
                                           approx=True, full_range=False),
            0.0)
        cc[0:_TILE, :] = (k2[0] * kact).astype(jnp.bfloat16)
        cc[_TILE:2 * _TILE, :] = (cin0[0] * k4[0] * kinh).astype(
            jnp.bfloat16)
        colsum = jnp.sum(kact, axis=0, keepdims=True) \
            + jnp.sum(kinh, axis=0, keepdims=True)

        @pl.when(t == 0)
        def _():
            v_ref[:] = colsum

        @pl.when(t > 0)
        def _():
            v_ref[:] = v_ref[:] + colsum

        dn = (((1,), (1,)), ((), ()))
        ab = jax.lax.dot_general(x_bf[:], cc[:], dn,
                                 preferred_element_type=jnp.float32)
        ai[:, pl.ds(2 * _TILE * t, 2 * _TILE)] = ab.astype(jnp.bfloat16)

    @pl.when(t == _T)
    def _finalize():
        acc = jnp.zeros((_B, 1), jnp.float32)
        for c in range(_D // _FC):
            sl = slice(c * _FC, (c + 1) * _FC)
            acc += jnp.sum(x_bf[:, sl].astype(jnp.float32) * v_ref[:, sl],
                           axis=1, keepdims=True)
        cp = out_ref[:] + acc
        cp2 = jnp.zeros((_B, 1), jnp.float32)
        for c in range(_D // _FC):
            sl = slice(c * _FC, (c + 1) * _FC)
            activ = ai[:, 2 * _FC * c:2 * _FC * c + _FC].astype(
                jnp.float32)
            inhib = ai[:, 2 * _FC * c + _FC:2 * _FC * (c + 1)].astype(
                jnp.float32)
            kdicp = kdi[0, :, sl] * cp
            num = _E0 * activ * kdicp
            den = kdt[0, :, sl] * kdicp * cp + _E0 * inhib + _EPS * kdicp
            x_eq = num * pl.reciprocal(den, approx=True, full_range=False)
            w5 = k5[0, :, sl] * pl.reciprocal(
                k5n[0, :, sl] + k6[0, :, sl] + _EPS,
                approx=True, full_range=False)
            cp2 += jnp.sum(x_eq * w5, axis=1, keepdims=True)
            x_bf[:, sl] = x_eq.astype(jnp.bfloat16)
        out_ref[:] = cp + cp2


def kernel(X0, k1, k1n, k2, k3, k3n, k4, TA0, TI0, Cinhib0, masks,
           k5, k5n, k6, kdI, kdT):
    big = pl.BlockSpec((1, _TILE, _D),
                       lambda l, t: (l, jnp.minimum(t, _T - 1), 0))
    vec = pl.BlockSpec((1, 1, _D), lambda l, t: (l, 0, 0))
    k5, k5n, k6, kdI, kdT = (a.reshape(_L, 1, _D)
                             for a in (k5, k5n, k6, kdI, kdT))
    cp = pl.pallas_call(
        _body,
        grid=(_L, _T + 1),
        in_specs=[pl.BlockSpec((_B, _D), lambda l, t: (0, 0))]
        + [big] * 10 + [vec] * 5,
        out_specs=pl.BlockSpec((_B, 1), lambda l, t: (0, 0)),
        out_shape=jax.ShapeDtypeStruct((_B, 1), jnp.float32),
        scratch_shapes=[
            pltpu.VMEM((_B, _D), jnp.bfloat16),
            pltpu.VMEM((_B, 2 * _D), jnp.bfloat16),
            pltpu.VMEM((2 * _TILE, _D), jnp.bfloat16),
            pltpu.VMEM((1, _D), jnp.float32),
        ],
        compiler_params=pltpu.CompilerParams(
            vmem_limit_bytes=100 * 1024 * 1024),
    )(X0, k1, k1n, k2, k3, k3n, k4, TA0, TI0, Cinhib0, masks,
      k5, k5n, k6, kdI, kdT)
    return cp.reshape(_B)
